# Initial kernel scaffold; baseline (speedup 1.0000x reference)
import jax
import jax.numpy as jnp
from jax.experimental import pallas as pl


def kernel(x, edge_index, W1, b1, W2, b2, W3, b3, Wl, bl):
    raise NotImplementedError("write your pallas kernel here")


# speedup vs baseline: 14.8390x; 14.8390x over previous
import jax
import jax.numpy as jnp
from jax import lax
from jax.experimental import pallas as pl
from jax.experimental.pallas import tpu as pltpu
from jax.experimental.pallas import tpu_sc as plsc

N = 100000
E = 1600000
H = 32

NC = 2
NS = 16
NHALF = N // NC
SPADR = 50176
TRASH = NHALF
ROWS_T = 3136
WCH = 392

BLK = 512
KR = 4
NBLK = 196
EPT = BLK * NBLK
EPAD = EPT * NS
ERows = EPAD // 128

_mesh = plsc.VectorSubcoreMesh(
    core_axis_name="c", subcore_axis_name="s", num_cores=NC, num_subcores=NS
)
_sc_params = pltpu.CompilerParams(use_tc_tiling_on_sc=False)


def _local_indices(dvm, livm, lo):
    def body(i, _):
        k = i >> 3
        j = (i & 7) * 16
        d = dvm[k, pl.ds(j, 16)]
        inr = (d >= lo) & (d < lo + NHALF)
        li = jnp.where(inr, d - lo, TRASH + (d & 127))
        livm[k, pl.ds(j, 16)] = li
        return 0

    lax.fori_loop(0, KR * 8, body, 0)


def _deg_body(dstp, deg_out, dvm, livm, ones_vm, zb, deg_sh):
    c = lax.axis_index("c")
    s = lax.axis_index("s")
    lo = c * NHALF

    ones16 = jnp.ones((16,), jnp.float32)
    for k in range(8):
        ones_vm[pl.ds(k * 16, 16)] = ones16

    def zbody(i, _):
        zb[pl.ds(i * 16, 16)] = jnp.zeros((16,), jnp.float32)
        return 0

    lax.fori_loop(0, ROWS_T // 16, zbody, 0)
    pltpu.sync_copy(zb, deg_sh.at[pl.ds(s * ROWS_T, ROWS_T)])
    plsc.subcore_barrier()

    def blk_body(blk, _):
        rowbase = s * (EPT // 128) + blk * KR
        pltpu.sync_copy(dstp.at[pl.ds(rowbase, KR)], dvm)
        _local_indices(dvm, livm, lo)
        for k in range(KR):
            pltpu.sync_copy(ones_vm, deg_sh.at[livm.at[k]], add=True)
        return 0

    lax.fori_loop(0, NBLK, blk_body, 0)
    plsc.subcore_barrier()
    pltpu.sync_copy(deg_sh.at[pl.ds(s * ROWS_T, ROWS_T)], zb)
    pltpu.sync_copy(zb, deg_out.at[pl.ds(c * SPADR + s * ROWS_T, ROWS_T)])


_deg_call = pl.kernel(
    _deg_body,
    out_type=jax.ShapeDtypeStruct((NC * SPADR,), jnp.float32),
    mesh=_mesh,
    scratch_types=[
        pltpu.VMEM((KR, 128), jnp.int32),
        pltpu.VMEM((KR, 128), jnp.int32),
        pltpu.VMEM((128,), jnp.float32),
        pltpu.VMEM((ROWS_T,), jnp.float32),
        pltpu.VMEM_SHARED((SPADR,), jnp.float32),
    ],
    compiler_params=_sc_params,
)


def _agg_body(z, srcp, dstp, agg_out, svm, dvm, livm, rows, acc_sh, gsem):
    c = lax.axis_index("c")
    s = lax.axis_index("s")
    lo = c * NHALF

    def zbody(r, _):
        z16 = jnp.zeros((16,), jnp.float32)
        rows[r, pl.ds(0, 16)] = z16
        rows[r, pl.ds(16, 16)] = z16
        return 0

    lax.fori_loop(0, WCH, zbody, 0)
    for q in range(8):
        pltpu.sync_copy(
            rows.at[pl.ds(0, WCH)], acc_sh.at[pl.ds(s * ROWS_T + q * WCH, WCH)]
        )
    plsc.subcore_barrier()

    def blk_body(blk, _):
        rowbase = s * (EPT // 128) + blk * KR
        pltpu.sync_copy(srcp.at[pl.ds(rowbase, KR)], svm)
        pltpu.sync_copy(dstp.at[pl.ds(rowbase, KR)], dvm)
        _local_indices(dvm, livm, lo)
        descs = [
            pltpu.async_copy(z.at[svm.at[k]], rows.at[pl.ds(k * 128, 128)], gsem)
            for k in range(KR)
        ]
        for d in descs:
            d.wait()
        for k in range(KR):
            pltpu.sync_copy(
                rows.at[pl.ds(k * 128, 128)], acc_sh.at[livm.at[k]], add=True
            )
        return 0

    lax.fori_loop(0, NBLK, blk_body, 0)
    plsc.subcore_barrier()
    for q in range(8):
        base = s * ROWS_T + q * WCH
        pltpu.sync_copy(acc_sh.at[pl.ds(base, WCH)], rows.at[pl.ds(0, WCH)])
        pltpu.sync_copy(
            rows.at[pl.ds(0, WCH)], agg_out.at[pl.ds(c * SPADR + base, WCH)]
        )


_agg_call = pl.kernel(
    _agg_body,
    out_type=jax.ShapeDtypeStruct((NC * SPADR, H), jnp.float32),
    mesh=_mesh,
    scratch_types=[
        pltpu.VMEM((KR, 128), jnp.int32),
        pltpu.VMEM((KR, 128), jnp.int32),
        pltpu.VMEM((KR, 128), jnp.int32),
        pltpu.VMEM((BLK, H), jnp.float32),
        pltpu.VMEM_SHARED((SPADR, H), jnp.float32),
        pltpu.SemaphoreType.DMA,
    ],
    compiler_params=_sc_params,
)


_RB = 1000
_GRID = N // _RB


def _dense0_body(deg_ref, x_ref, w_ref, dinv_ref, z_ref):
    dinv = lax.rsqrt(deg_ref[...] + 1.0)
    dinv_ref[...] = dinv
    z_ref[...] = dinv * jnp.dot(
        x_ref[...], w_ref[...], preferred_element_type=jnp.float32
    )


_dense0 = pl.pallas_call(
    _dense0_body,
    grid=(_GRID,),
    in_specs=[
        pl.BlockSpec((_RB, 1), lambda i: (i, 0)),
        pl.BlockSpec((_RB, 2), lambda i: (i, 0)),
        pl.BlockSpec((2, H), lambda i: (0, 0)),
    ],
    out_specs=[
        pl.BlockSpec((_RB, 1), lambda i: (i, 0)),
        pl.BlockSpec((_RB, H), lambda i: (i, 0)),
    ],
    out_shape=[
        jax.ShapeDtypeStruct((N, 1), jnp.float32),
        jax.ShapeDtypeStruct((N, H), jnp.float32),
    ],
)


def _dense_mid_body(agg_ref, z_ref, dinv_ref, b_ref, w_ref, zn_ref):
    dinv = dinv_ref[...]
    h = jnp.maximum(dinv * (agg_ref[...] + z_ref[...]) + b_ref[...], 0.0)
    zn_ref[...] = dinv * jnp.dot(
        h, w_ref[...], preferred_element_type=jnp.float32
    )


_dense_mid = pl.pallas_call(
    _dense_mid_body,
    grid=(_GRID,),
    in_specs=[
        pl.BlockSpec((_RB, H), lambda i: (i, 0)),
        pl.BlockSpec((_RB, H), lambda i: (i, 0)),
        pl.BlockSpec((_RB, 1), lambda i: (i, 0)),
        pl.BlockSpec((1, H), lambda i: (0, 0)),
        pl.BlockSpec((H, H), lambda i: (0, 0)),
    ],
    out_specs=pl.BlockSpec((_RB, H), lambda i: (i, 0)),
    out_shape=jax.ShapeDtypeStruct((N, H), jnp.float32),
)


def _dense_fin_body(agg_ref, z_ref, dinv_ref, b_ref, wl_ref, bl_ref, out_ref):
    dinv = dinv_ref[...]
    h = jnp.maximum(dinv * (agg_ref[...] + z_ref[...]) + b_ref[...], 0.0)
    out_ref[...] = (
        jnp.dot(h, wl_ref[...], preferred_element_type=jnp.float32) + bl_ref[...]
    )


_dense_fin = pl.pallas_call(
    _dense_fin_body,
    grid=(_GRID,),
    in_specs=[
        pl.BlockSpec((_RB, H), lambda i: (i, 0)),
        pl.BlockSpec((_RB, H), lambda i: (i, 0)),
        pl.BlockSpec((_RB, 1), lambda i: (i, 0)),
        pl.BlockSpec((1, H), lambda i: (0, 0)),
        pl.BlockSpec((H, 1), lambda i: (0, 0)),
        pl.BlockSpec((1, 1), lambda i: (0, 0)),
    ],
    out_specs=pl.BlockSpec((_RB, 1), lambda i: (i, 0)),
    out_shape=jax.ShapeDtypeStruct((N, 1), jnp.float32),
)


def _unpad(a):
    return jnp.concatenate([a[:NHALF], a[SPADR : SPADR + NHALF]], axis=0)


@jax.jit
def kernel(x, edge_index, W1, b1, W2, b2, W3, b3, Wl, bl):
    src = edge_index[0]
    dst = edge_index[1]
    pad = EPAD - E
    srcp = jnp.concatenate([src, jnp.zeros((pad,), jnp.int32)]).reshape(ERows, 128)
    dstp = jnp.concatenate([dst, jnp.full((pad,), N, jnp.int32)]).reshape(ERows, 128)

    degp = _deg_call(dstp)
    deg = _unpad(degp)[:, None]

    dinv, z1 = _dense0(deg, x, W1)
    a1 = _unpad(_agg_call(z1, srcp, dstp))
    z2 = _dense_mid(a1, z1, dinv, b1[None, :], W2)
    a2 = _unpad(_agg_call(z2, srcp, dstp))
    z3 = _dense_mid(a2, z2, dinv, b2[None, :], W3)
    a3 = _unpad(_agg_call(z3, srcp, dstp))
    return _dense_fin(a3, z3, dinv, b3[None, :], Wl, bl[None, :])


# speedup vs baseline: 16.5672x; 1.1165x over previous
import jax
import jax.numpy as jnp
from jax import lax
from jax.experimental import pallas as pl
from jax.experimental.pallas import tpu as pltpu
from jax.experimental.pallas import tpu_sc as plsc

N = 100000
E = 1600000
H = 32

NC = 2
NS = 16
NHALF = N // NC
SPADR = 50176
TRASH = NHALF
ROWS_T = 3136
WCH = 224

KR = 3
BLK = KR * 128
NBLK = 262
EPT = BLK * NBLK
EPAD = EPT * NS
ERows = EPAD // 128
RPT = EPT // 128

_mesh = plsc.VectorSubcoreMesh(
    core_axis_name="c", subcore_axis_name="s", num_cores=NC, num_subcores=NS
)
_sc_params = pltpu.CompilerParams(use_tc_tiling_on_sc=False)


def _local_indices(dvm, livm, lo):
    def body(i, _):
        k = i >> 3
        j = (i & 7) * 16
        d = dvm[k, pl.ds(j, 16)]
        inr = (d >= lo) & (d < lo + NHALF)
        li = jnp.where(inr, d - lo, TRASH + (d & 127))
        livm[k, pl.ds(j, 16)] = li
        return 0

    lax.fori_loop(0, KR * 8, body, 0)


def _deg_body(dstp, deg_out, dvm_a, dvm_b, livm_a, livm_b, ones_vm, zb,
              isem_a, isem_b, ssem_a, ssem_b, deg_sh):
    c = lax.axis_index("c")
    s = lax.axis_index("s")
    lo = c * NHALF
    rbase = s * RPT

    ones16 = jnp.ones((16,), jnp.float32)
    for k in range(8):
        ones_vm[pl.ds(k * 16, 16)] = ones16

    def zbody(i, _):
        zb[pl.ds(i * 16, 16)] = jnp.zeros((16,), jnp.float32)
        return 0

    lax.fori_loop(0, ROWS_T // 16, zbody, 0)
    pltpu.sync_copy(zb, deg_sh.at[pl.ds(s * ROWS_T, ROWS_T)])
    plsc.subcore_barrier()

    pltpu.async_copy(dstp.at[pl.ds(rbase, KR)], dvm_a, isem_a)

    def step(b, dvm, livm, dvm_n, isem, isem_n, ssem):
        pltpu.make_async_copy(dstp.at[pl.ds(rbase, KR)], dvm, isem).wait()

        @pl.when(b + 1 < NBLK)
        def _():
            pltpu.async_copy(
                dstp.at[pl.ds(rbase + (b + 1) * KR, KR)], dvm_n, isem_n
            )

        _local_indices(dvm, livm, lo)
        for k in range(KR):
            pltpu.sync_copy(ones_vm, deg_sh.at[livm.at[k]], add=True)

    def pair(i, _):
        step(2 * i, dvm_a, livm_a, dvm_b, isem_a, isem_b, ssem_a)
        step(2 * i + 1, dvm_b, livm_b, dvm_a, isem_b, isem_a, ssem_b)
        return 0

    lax.fori_loop(0, NBLK // 2, pair, 0)

    plsc.subcore_barrier()
    pltpu.sync_copy(deg_sh.at[pl.ds(s * ROWS_T, ROWS_T)], zb)
    pltpu.sync_copy(zb, deg_out.at[pl.ds(c * SPADR + s * ROWS_T, ROWS_T)])


_deg_call = pl.kernel(
    _deg_body,
    out_type=jax.ShapeDtypeStruct((NC * SPADR,), jnp.float32),
    mesh=_mesh,
    scratch_types=[
        pltpu.VMEM((KR, 128), jnp.int32),
        pltpu.VMEM((KR, 128), jnp.int32),
        pltpu.VMEM((KR, 128), jnp.int32),
        pltpu.VMEM((KR, 128), jnp.int32),
        pltpu.VMEM((128,), jnp.float32),
        pltpu.VMEM((ROWS_T,), jnp.float32),
        pltpu.SemaphoreType.DMA,
        pltpu.SemaphoreType.DMA,
        pltpu.SemaphoreType.DMA,
        pltpu.SemaphoreType.DMA,
        pltpu.VMEM_SHARED((SPADR,), jnp.float32),
    ],
    compiler_params=_sc_params,
)


def _agg_body(z, srcp, dstp, agg_out,
              svm_a, svm_b, dvm_a, dvm_b, livm_a, livm_b, rows_a, rows_b,
              isem_a, isem_b, gsem, ssem_a, ssem_b, acc_sh):
    c = lax.axis_index("c")
    s = lax.axis_index("s")
    lo = c * NHALF
    rbase = s * RPT

    def zbody(r, _):
        z16 = jnp.zeros((16,), jnp.float32)
        rows_a[r, pl.ds(0, 16)] = z16
        rows_a[r, pl.ds(16, 16)] = z16
        return 0

    lax.fori_loop(0, WCH, zbody, 0)
    for q in range(ROWS_T // WCH):
        pltpu.sync_copy(
            rows_a.at[pl.ds(0, WCH)], acc_sh.at[pl.ds(s * ROWS_T + q * WCH, WCH)]
        )
    plsc.subcore_barrier()

    pltpu.async_copy(srcp.at[pl.ds(rbase, KR)], svm_a, isem_a)
    pltpu.async_copy(dstp.at[pl.ds(rbase, KR)], dvm_a, isem_a)

    def step(b, svm, dvm, livm, rows, svm_n, dvm_n, isem, isem_n, ssem):
        pltpu.make_async_copy(srcp.at[pl.ds(rbase, KR)], svm, isem).wait()
        pltpu.make_async_copy(dstp.at[pl.ds(rbase, KR)], dvm, isem).wait()

        descs = [
            pltpu.async_copy(z.at[svm.at[k]], rows.at[pl.ds(k * 128, 128)], gsem)
            for k in range(KR)
        ]

        @pl.when(b + 1 < NBLK)
        def _():
            nb = rbase + (b + 1) * KR
            pltpu.async_copy(srcp.at[pl.ds(nb, KR)], svm_n, isem_n)
            pltpu.async_copy(dstp.at[pl.ds(nb, KR)], dvm_n, isem_n)

        _local_indices(dvm, livm, lo)

        for d in descs:
            d.wait()

        for k in range(KR):
            pltpu.sync_copy(
                rows.at[pl.ds(k * 128, 128)], acc_sh.at[livm.at[k]], add=True
            )

    def pair(i, _):
        step(2 * i, svm_a, dvm_a, livm_a, rows_a, svm_b, dvm_b,
             isem_a, isem_b, ssem_a)
        step(2 * i + 1, svm_b, dvm_b, livm_b, rows_b, svm_a, dvm_a,
             isem_b, isem_a, ssem_b)
        return 0

    lax.fori_loop(0, NBLK // 2, pair, 0)

    plsc.subcore_barrier()
    for q in range(ROWS_T // WCH):
        base = s * ROWS_T + q * WCH
        pltpu.sync_copy(acc_sh.at[pl.ds(base, WCH)], rows_a.at[pl.ds(0, WCH)])
        pltpu.sync_copy(
            rows_a.at[pl.ds(0, WCH)], agg_out.at[pl.ds(c * SPADR + base, WCH)]
        )


_agg_call = pl.kernel(
    _agg_body,
    out_type=jax.ShapeDtypeStruct((NC * SPADR, H), jnp.float32),
    mesh=_mesh,
    scratch_types=[
        pltpu.VMEM((KR, 128), jnp.int32),
        pltpu.VMEM((KR, 128), jnp.int32),
        pltpu.VMEM((KR, 128), jnp.int32),
        pltpu.VMEM((KR, 128), jnp.int32),
        pltpu.VMEM((KR, 128), jnp.int32),
        pltpu.VMEM((KR, 128), jnp.int32),
        pltpu.VMEM((BLK, H), jnp.float32),
        pltpu.VMEM((BLK, H), jnp.float32),
        pltpu.SemaphoreType.DMA,
        pltpu.SemaphoreType.DMA,
        pltpu.SemaphoreType.DMA,
        pltpu.SemaphoreType.DMA,
        pltpu.SemaphoreType.DMA,
        pltpu.VMEM_SHARED((SPADR, H), jnp.float32),
    ],
    compiler_params=_sc_params,
)


_RB = 1000
_GRID = N // _RB


def _dense0_body(deg_ref, x_ref, w_ref, dinv_ref, z_ref):
    dinv = lax.rsqrt(deg_ref[...] + 1.0)
    dinv_ref[...] = dinv
    z_ref[...] = dinv * jnp.dot(
        x_ref[...], w_ref[...], preferred_element_type=jnp.float32
    )


_dense0 = pl.pallas_call(
    _dense0_body,
    grid=(_GRID,),
    in_specs=[
        pl.BlockSpec((_RB, 1), lambda i: (i, 0)),
        pl.BlockSpec((_RB, 2), lambda i: (i, 0)),
        pl.BlockSpec((2, H), lambda i: (0, 0)),
    ],
    out_specs=[
        pl.BlockSpec((_RB, 1), lambda i: (i, 0)),
        pl.BlockSpec((_RB, H), lambda i: (i, 0)),
    ],
    out_shape=[
        jax.ShapeDtypeStruct((N, 1), jnp.float32),
        jax.ShapeDtypeStruct((N, H), jnp.float32),
    ],
)


def _dense_mid_body(agg_ref, z_ref, dinv_ref, b_ref, w_ref, zn_ref):
    dinv = dinv_ref[...]
    h = jnp.maximum(dinv * (agg_ref[...] + z_ref[...]) + b_ref[...], 0.0)
    zn_ref[...] = dinv * jnp.dot(
        h, w_ref[...], preferred_element_type=jnp.float32
    )


_dense_mid = pl.pallas_call(
    _dense_mid_body,
    grid=(_GRID,),
    in_specs=[
        pl.BlockSpec((_RB, H), lambda i: (i, 0)),
        pl.BlockSpec((_RB, H), lambda i: (i, 0)),
        pl.BlockSpec((_RB, 1), lambda i: (i, 0)),
        pl.BlockSpec((1, H), lambda i: (0, 0)),
        pl.BlockSpec((H, H), lambda i: (0, 0)),
    ],
    out_specs=pl.BlockSpec((_RB, H), lambda i: (i, 0)),
    out_shape=jax.ShapeDtypeStruct((N, H), jnp.float32),
)


def _dense_fin_body(agg_ref, z_ref, dinv_ref, b_ref, wl_ref, bl_ref, out_ref):
    dinv = dinv_ref[...]
    h = jnp.maximum(dinv * (agg_ref[...] + z_ref[...]) + b_ref[...], 0.0)
    out_ref[...] = (
        jnp.dot(h, wl_ref[...], preferred_element_type=jnp.float32) + bl_ref[...]
    )


_dense_fin = pl.pallas_call(
    _dense_fin_body,
    grid=(_GRID,),
    in_specs=[
        pl.BlockSpec((_RB, H), lambda i: (i, 0)),
        pl.BlockSpec((_RB, H), lambda i: (i, 0)),
        pl.BlockSpec((_RB, 1), lambda i: (i, 0)),
        pl.BlockSpec((1, H), lambda i: (0, 0)),
        pl.BlockSpec((H, 1), lambda i: (0, 0)),
        pl.BlockSpec((1, 1), lambda i: (0, 0)),
    ],
    out_specs=pl.BlockSpec((_RB, 1), lambda i: (i, 0)),
    out_shape=jax.ShapeDtypeStruct((N, 1), jnp.float32),
)


def _unpad(a):
    return jnp.concatenate([a[:NHALF], a[SPADR : SPADR + NHALF]], axis=0)


@jax.jit
def kernel(x, edge_index, W1, b1, W2, b2, W3, b3, Wl, bl):
    src = edge_index[0]
    dst = edge_index[1]
    pad = EPAD - E
    srcp = jnp.concatenate([src, jnp.zeros((pad,), jnp.int32)]).reshape(ERows, 128)
    dstp = jnp.concatenate([dst, jnp.full((pad,), N, jnp.int32)]).reshape(ERows, 128)

    degp = _deg_call(dstp)
    deg = _unpad(degp)[:, None]

    dinv, z1 = _dense0(deg, x, W1)
    a1 = _unpad(_agg_call(z1, srcp, dstp))
    z2 = _dense_mid(a1, z1, dinv, b1[None, :], W2)
    a2 = _unpad(_agg_call(z2, srcp, dstp))
    z3 = _dense_mid(a2, z2, dinv, b2[None, :], W3)
    a3 = _unpad(_agg_call(z3, srcp, dstp))
    return _dense_fin(a3, z3, dinv, b3[None, :], Wl, bl[None, :])


# speedup vs baseline: 18.5335x; 1.1187x over previous
import jax
import jax.numpy as jnp
from jax import lax
from jax.experimental import pallas as pl
from jax.experimental.pallas import tpu as pltpu
from jax.experimental.pallas import tpu_sc as plsc

N = 100000
E = 1600000
H = 32

NC = 2
NS = 16
NHALF = N // NC
SPADR = 50176
TRASH = NHALF
ROWS_T = 3136
WCH = 224

KR = 3
BLK = KR * 128
NBLK = 262
EPT = BLK * NBLK
EPAD = EPT * NS
ERows = EPAD // 128
RPT = EPT // 128

_mesh = plsc.VectorSubcoreMesh(
    core_axis_name="c", subcore_axis_name="s", num_cores=NC, num_subcores=NS
)
_sc_params = pltpu.CompilerParams(use_tc_tiling_on_sc=False)


def _local_indices(dvm, livm, lo):
    def body(i, _):
        k = i >> 3
        j = (i & 7) * 16
        d = dvm[k, pl.ds(j, 16)]
        inr = (d >= lo) & (d < lo + NHALF)
        li = jnp.where(inr, d - lo, TRASH + (d & 127))
        livm[k, pl.ds(j, 16)] = li
        return 0

    lax.fori_loop(0, KR * 8, body, 0)


def _deg_body(dstp, deg_out, dvm_a, dvm_b, livm_a, livm_b, ones_vm, zb,
              isem_a, isem_b, ssem_a, ssem_b, deg_sh):
    c = lax.axis_index("c")
    s = lax.axis_index("s")
    lo = c * NHALF
    rbase = s * RPT

    ones16 = jnp.ones((16,), jnp.float32)
    for k in range(8):
        ones_vm[pl.ds(k * 16, 16)] = ones16

    def zbody(i, _):
        zb[pl.ds(i * 16, 16)] = jnp.zeros((16,), jnp.float32)
        return 0

    lax.fori_loop(0, ROWS_T // 16, zbody, 0)
    pltpu.sync_copy(zb, deg_sh.at[pl.ds(s * ROWS_T, ROWS_T)])
    plsc.subcore_barrier()

    pltpu.async_copy(dstp.at[pl.ds(rbase, KR)], dvm_a, isem_a)

    def step(b, dvm, livm, dvm_n, isem, isem_n, ssem):
        pltpu.make_async_copy(dstp.at[pl.ds(rbase, KR)], dvm, isem).wait()

        @pl.when(b + 1 < NBLK)
        def _():
            pltpu.async_copy(
                dstp.at[pl.ds(rbase + (b + 1) * KR, KR)], dvm_n, isem_n
            )

        _local_indices(dvm, livm, lo)
        for k in range(KR):
            pltpu.sync_copy(ones_vm, deg_sh.at[livm.at[k]], add=True)

    def pair(i, _):
        step(2 * i, dvm_a, livm_a, dvm_b, isem_a, isem_b, ssem_a)
        step(2 * i + 1, dvm_b, livm_b, dvm_a, isem_b, isem_a, ssem_b)
        return 0

    lax.fori_loop(0, NBLK // 2, pair, 0)

    plsc.subcore_barrier()
    pltpu.sync_copy(deg_sh.at[pl.ds(s * ROWS_T, ROWS_T)], zb)
    pltpu.sync_copy(zb, deg_out.at[pl.ds(c * SPADR + s * ROWS_T, ROWS_T)])


_deg_call = pl.kernel(
    _deg_body,
    out_type=jax.ShapeDtypeStruct((NC * SPADR,), jnp.float32),
    mesh=_mesh,
    scratch_types=[
        pltpu.VMEM((KR, 128), jnp.int32),
        pltpu.VMEM((KR, 128), jnp.int32),
        pltpu.VMEM((KR, 128), jnp.int32),
        pltpu.VMEM((KR, 128), jnp.int32),
        pltpu.VMEM((128,), jnp.float32),
        pltpu.VMEM((ROWS_T,), jnp.float32),
        pltpu.SemaphoreType.DMA,
        pltpu.SemaphoreType.DMA,
        pltpu.SemaphoreType.DMA,
        pltpu.SemaphoreType.DMA,
        pltpu.VMEM_SHARED((SPADR,), jnp.float32),
    ],
    compiler_params=_sc_params,
)


def _agg_body(z, srcp, dstp, agg_out,
              svm_a, svm_b, dvm_a, dvm_b, livm_a, livm_b, rows_a, rows_b,
              isem_a, isem_b, gsem, ssem_a, ssem_b, acc_sh):
    c = lax.axis_index("c")
    s = lax.axis_index("s")
    lo = c * NHALF
    rbase = s * RPT

    def zbody(r, _):
        z16 = jnp.zeros((16,), jnp.float32)
        rows_a[r, pl.ds(0, 16)] = z16
        rows_a[r, pl.ds(16, 16)] = z16
        return 0

    lax.fori_loop(0, WCH, zbody, 0)
    for q in range(ROWS_T // WCH):
        pltpu.sync_copy(
            rows_a.at[pl.ds(0, WCH)], acc_sh.at[pl.ds(s * ROWS_T + q * WCH, WCH)]
        )
    plsc.subcore_barrier()

    pltpu.async_copy(srcp.at[pl.ds(rbase, KR)], svm_a, isem_a)
    pltpu.async_copy(dstp.at[pl.ds(rbase, KR)], dvm_a, isem_a)

    def wait_idx(svm, dvm, isem):
        pltpu.make_async_copy(srcp.at[pl.ds(rbase, KR)], svm, isem).wait()
        pltpu.make_async_copy(dstp.at[pl.ds(rbase, KR)], dvm, isem).wait()

    def fire_gathers(svm, rows):
        return [
            pltpu.async_copy(z.at[svm.at[k]], rows.at[pl.ds(k * 128, 128)], gsem)
            for k in range(KR)
        ]

    def fire_idx(b, svm, dvm, isem):
        @pl.when(b < NBLK)
        def _():
            nb = rbase + b * KR
            pltpu.async_copy(srcp.at[pl.ds(nb, KR)], svm, isem)
            pltpu.async_copy(dstp.at[pl.ds(nb, KR)], dvm, isem)

    def scatter(livm, rows):
        for k in range(KR):
            pltpu.sync_copy(
                rows.at[pl.ds(k * 128, 128)], acc_sh.at[livm.at[k]], add=True
            )

    def pair(i, _):
        b0 = 2 * i
        wait_idx(svm_a, dvm_a, isem_a)
        ga = fire_gathers(svm_a, rows_a)
        wait_idx(svm_b, dvm_b, isem_b)
        gb = fire_gathers(svm_b, rows_b)
        _local_indices(dvm_a, livm_a, lo)
        for d in ga:
            d.wait()
        fire_idx(b0 + 2, svm_a, dvm_a, isem_a)
        scatter(livm_a, rows_a)
        _local_indices(dvm_b, livm_b, lo)
        for d in gb:
            d.wait()
        fire_idx(b0 + 3, svm_b, dvm_b, isem_b)
        scatter(livm_b, rows_b)
        return 0

    pltpu.async_copy(srcp.at[pl.ds(rbase + KR, KR)], svm_b, isem_b)
    pltpu.async_copy(dstp.at[pl.ds(rbase + KR, KR)], dvm_b, isem_b)
    lax.fori_loop(0, NBLK // 2, pair, 0)

    plsc.subcore_barrier()
    for q in range(ROWS_T // WCH):
        base = s * ROWS_T + q * WCH
        pltpu.sync_copy(acc_sh.at[pl.ds(base, WCH)], rows_a.at[pl.ds(0, WCH)])
        pltpu.sync_copy(
            rows_a.at[pl.ds(0, WCH)], agg_out.at[pl.ds(c * SPADR + base, WCH)]
        )


_agg_call = pl.kernel(
    _agg_body,
    out_type=jax.ShapeDtypeStruct((NC * SPADR, H), jnp.float32),
    mesh=_mesh,
    scratch_types=[
        pltpu.VMEM((KR, 128), jnp.int32),
        pltpu.VMEM((KR, 128), jnp.int32),
        pltpu.VMEM((KR, 128), jnp.int32),
        pltpu.VMEM((KR, 128), jnp.int32),
        pltpu.VMEM((KR, 128), jnp.int32),
        pltpu.VMEM((KR, 128), jnp.int32),
        pltpu.VMEM((BLK, H), jnp.float32),
        pltpu.VMEM((BLK, H), jnp.float32),
        pltpu.SemaphoreType.DMA,
        pltpu.SemaphoreType.DMA,
        pltpu.SemaphoreType.DMA,
        pltpu.SemaphoreType.DMA,
        pltpu.SemaphoreType.DMA,
        pltpu.VMEM_SHARED((SPADR, H), jnp.float32),
    ],
    compiler_params=_sc_params,
)


_RB = 1000
_GRID = N // _RB


def _dense0_body(deg_ref, x_ref, w_ref, dinv_ref, z_ref):
    dinv = lax.rsqrt(deg_ref[...] + 1.0)
    dinv_ref[...] = dinv
    z_ref[...] = dinv * jnp.dot(
        x_ref[...], w_ref[...], preferred_element_type=jnp.float32
    )


_dense0 = pl.pallas_call(
    _dense0_body,
    grid=(_GRID,),
    in_specs=[
        pl.BlockSpec((_RB, 1), lambda i: (i, 0)),
        pl.BlockSpec((_RB, 2), lambda i: (i, 0)),
        pl.BlockSpec((2, H), lambda i: (0, 0)),
    ],
    out_specs=[
        pl.BlockSpec((_RB, 1), lambda i: (i, 0)),
        pl.BlockSpec((_RB, H), lambda i: (i, 0)),
    ],
    out_shape=[
        jax.ShapeDtypeStruct((N, 1), jnp.float32),
        jax.ShapeDtypeStruct((N, H), jnp.float32),
    ],
)


def _dense_mid_body(agg_ref, z_ref, dinv_ref, b_ref, w_ref, zn_ref):
    dinv = dinv_ref[...]
    h = jnp.maximum(dinv * (agg_ref[...] + z_ref[...]) + b_ref[...], 0.0)
    zn_ref[...] = dinv * jnp.dot(
        h, w_ref[...], preferred_element_type=jnp.float32
    )


_dense_mid = pl.pallas_call(
    _dense_mid_body,
    grid=(_GRID,),
    in_specs=[
        pl.BlockSpec((_RB, H), lambda i: (i, 0)),
        pl.BlockSpec((_RB, H), lambda i: (i, 0)),
        pl.BlockSpec((_RB, 1), lambda i: (i, 0)),
        pl.BlockSpec((1, H), lambda i: (0, 0)),
        pl.BlockSpec((H, H), lambda i: (0, 0)),
    ],
    out_specs=pl.BlockSpec((_RB, H), lambda i: (i, 0)),
    out_shape=jax.ShapeDtypeStruct((N, H), jnp.float32),
)


def _dense_fin_body(agg_ref, z_ref, dinv_ref, b_ref, wl_ref, bl_ref, out_ref):
    dinv = dinv_ref[...]
    h = jnp.maximum(dinv * (agg_ref[...] + z_ref[...]) + b_ref[...], 0.0)
    out_ref[...] = (
        jnp.dot(h, wl_ref[...], preferred_element_type=jnp.float32) + bl_ref[...]
    )


_dense_fin = pl.pallas_call(
    _dense_fin_body,
    grid=(_GRID,),
    in_specs=[
        pl.BlockSpec((_RB, H), lambda i: (i, 0)),
        pl.BlockSpec((_RB, H), lambda i: (i, 0)),
        pl.BlockSpec((_RB, 1), lambda i: (i, 0)),
        pl.BlockSpec((1, H), lambda i: (0, 0)),
        pl.BlockSpec((H, 1), lambda i: (0, 0)),
        pl.BlockSpec((1, 1), lambda i: (0, 0)),
    ],
    out_specs=pl.BlockSpec((_RB, 1), lambda i: (i, 0)),
    out_shape=jax.ShapeDtypeStruct((N, 1), jnp.float32),
)


def _unpad(a):
    return jnp.concatenate([a[:NHALF], a[SPADR : SPADR + NHALF]], axis=0)


@jax.jit
def kernel(x, edge_index, W1, b1, W2, b2, W3, b3, Wl, bl):
    src = edge_index[0]
    dst = edge_index[1]
    pad = EPAD - E
    srcp = jnp.concatenate([src, jnp.zeros((pad,), jnp.int32)]).reshape(ERows, 128)
    dstp = jnp.concatenate([dst, jnp.full((pad,), N, jnp.int32)]).reshape(ERows, 128)

    degp = _deg_call(dstp)
    deg = _unpad(degp)[:, None]

    dinv, z1 = _dense0(deg, x, W1)
    a1 = _unpad(_agg_call(z1, srcp, dstp))
    z2 = _dense_mid(a1, z1, dinv, b1[None, :], W2)
    a2 = _unpad(_agg_call(z2, srcp, dstp))
    z3 = _dense_mid(a2, z2, dinv, b2[None, :], W3)
    a3 = _unpad(_agg_call(z3, srcp, dstp))
    return _dense_fin(a3, z3, dinv, b3[None, :], Wl, bl[None, :])


# speedup vs baseline: 25.0334x; 1.3507x over previous
import jax
import jax.numpy as jnp
from jax import lax
from jax.experimental import pallas as pl
from jax.experimental.pallas import tpu as pltpu
from jax.experimental.pallas import tpu_sc as plsc

N = 100000
E = 1600000
H = 32

NC = 2
NS = 16
NHALF = N // NC
SPADR = 50176
TRASH = NHALF
ROWS_T = 3136
WCH = 224

KR = 3
BLK = KR * 128
NBLK = 262
EPT = BLK * NBLK
EPAD = EPT * NS
ERows = EPAD // 128
RPT = EPT // 128

NBINB = NBLK + 2
ETile = NBINB * BLK
BRows = ETile // 128
NW = NC * NS

_mesh = plsc.VectorSubcoreMesh(
    core_axis_name="c", subcore_axis_name="s", num_cores=NC, num_subcores=NS
)
_sc_params = pltpu.CompilerParams(
    use_tc_tiling_on_sc=False, needs_layout_passes=False
)


def _local_indices(dvm, livm, lo):
    def body(i, _):
        k = i >> 3
        j = (i & 7) * 16
        d = dvm[k, pl.ds(j, 16)]
        inr = (d >= lo) & (d < lo + NHALF)
        li = jnp.where(inr, d - lo, TRASH + (d & 127))
        livm[k, pl.ds(j, 16)] = li
        return 0

    lax.fori_loop(0, KR * 8, body, 0)


def _deg_body(dstp, deg_out, dvm_a, dvm_b, livm_a, livm_b, ones_vm, zb,
              isem_a, isem_b, ssem_a, ssem_b, deg_sh):
    c = lax.axis_index("c")
    s = lax.axis_index("s")
    lo = c * NHALF
    rbase = s * RPT

    ones16 = jnp.ones((16,), jnp.float32)
    for k in range(8):
        ones_vm[pl.ds(k * 16, 16)] = ones16

    def zbody(i, _):
        zb[pl.ds(i * 16, 16)] = jnp.zeros((16,), jnp.float32)
        return 0

    lax.fori_loop(0, ROWS_T // 16, zbody, 0)
    pltpu.sync_copy(zb, deg_sh.at[pl.ds(s * ROWS_T, ROWS_T)])
    plsc.subcore_barrier()

    pltpu.async_copy(dstp.at[pl.ds(rbase, KR)], dvm_a, isem_a)

    def step(b, dvm, livm, dvm_n, isem, isem_n, ssem):
        pltpu.make_async_copy(dstp.at[pl.ds(rbase, KR)], dvm, isem).wait()

        @pl.when(b + 1 < NBLK)
        def _():
            pltpu.async_copy(
                dstp.at[pl.ds(rbase + (b + 1) * KR, KR)], dvm_n, isem_n
            )

        _local_indices(dvm, livm, lo)
        for k in range(KR):
            pltpu.sync_copy(ones_vm, deg_sh.at[livm.at[k]], add=True)

    def pair(i, _):
        step(2 * i, dvm_a, livm_a, dvm_b, isem_a, isem_b, ssem_a)
        step(2 * i + 1, dvm_b, livm_b, dvm_a, isem_b, isem_a, ssem_b)
        return 0

    lax.fori_loop(0, NBLK // 2, pair, 0)

    plsc.subcore_barrier()
    pltpu.sync_copy(deg_sh.at[pl.ds(s * ROWS_T, ROWS_T)], zb)
    pltpu.sync_copy(zb, deg_out.at[pl.ds(c * SPADR + s * ROWS_T, ROWS_T)])


_deg_call = pl.kernel(
    _deg_body,
    out_type=jax.ShapeDtypeStruct((NC * SPADR,), jnp.float32),
    mesh=_mesh,
    scratch_types=[
        pltpu.VMEM((KR, 128), jnp.int32),
        pltpu.VMEM((KR, 128), jnp.int32),
        pltpu.VMEM((KR, 128), jnp.int32),
        pltpu.VMEM((KR, 128), jnp.int32),
        pltpu.VMEM((128,), jnp.float32),
        pltpu.VMEM((ROWS_T,), jnp.float32),
        pltpu.SemaphoreType.DMA,
        pltpu.SemaphoreType.DMA,
        pltpu.SemaphoreType.DMA,
        pltpu.SemaphoreType.DMA,
        pltpu.VMEM_SHARED((SPADR,), jnp.float32),
    ],
    compiler_params=_sc_params,
)


def _bin_body(srcp, dstp, sbin, lbin, cnts,
              svm_a, svm_b, dvm_a, dvm_b, s_st, l_st, cvm, isem_a, isem_b):
    c = lax.axis_index("c")
    s = lax.axis_index("s")
    lo = c * NHALF
    rbase = s * RPT
    w = c * NS + s
    obase = w * ETile

    pltpu.async_copy(srcp.at[pl.ds(rbase, KR)], svm_a, isem_a)
    pltpu.async_copy(dstp.at[pl.ds(rbase, KR)], dvm_a, isem_a)
    pltpu.async_copy(srcp.at[pl.ds(rbase + KR, KR)], svm_b, isem_b)
    pltpu.async_copy(dstp.at[pl.ds(rbase + KR, KR)], dvm_b, isem_b)

    def flush(nf):
        pltpu.sync_copy(s_st.at[pl.ds(0, BLK)],
                        sbin.at[pl.ds(obase + nf * BLK, BLK)])
        pltpu.sync_copy(l_st.at[pl.ds(0, BLK)],
                        lbin.at[pl.ds(obase + nf * BLK, BLK)])
        for j in range(BLK // 16):
            s_st[pl.ds(16 * j, 16)] = s_st[pl.ds(BLK + 16 * j, 16)]
            l_st[pl.ds(16 * j, 16)] = l_st[pl.ds(BLK + 16 * j, 16)]

    def step(b, svm, dvm, isem, carry):
        sp, nf = carry
        pltpu.make_async_copy(srcp.at[pl.ds(rbase, KR)], svm, isem).wait()
        pltpu.make_async_copy(dstp.at[pl.ds(rbase, KR)], dvm, isem).wait()

        lane = lax.broadcasted_iota(jnp.int32, (16,), 0)

        def vbody(i, sp):
            k = i >> 3
            j = (i & 7) * 16
            d = dvm[k, pl.ds(j, 16)]
            sv = svm[k, pl.ds(j, 16)]
            inr = (d >= lo) & (d < lo + NHALF)
            key = jnp.where(inr, lane, lane + 16)
            ld = jnp.where(inr, d - lo, TRASH + (d & 127))
            _, sv_c = plsc.sort_key_val(key, sv)
            _, ld_c = plsc.sort_key_val(key, ld)
            s_st[pl.ds(sp, 16)] = sv_c
            l_st[pl.ds(sp, 16)] = ld_c
            pc = lax.reduce_sum(jnp.where(inr, 1, 0).astype(jnp.int32), (0,))
            return sp + pc

        sp = lax.fori_loop(0, KR * 8, vbody, sp)

        @pl.when(b + 2 < NBLK)
        def _():
            nb2 = rbase + (b + 2) * KR
            pltpu.async_copy(srcp.at[pl.ds(nb2, KR)], svm, isem)
            pltpu.async_copy(dstp.at[pl.ds(nb2, KR)], dvm, isem)

        do_flush = sp >= BLK

        @pl.when(do_flush)
        def _():
            flush(nf)

        sp = jnp.where(do_flush, sp - BLK, sp)
        nf = jnp.where(do_flush, nf + 1, nf)
        return sp, nf

    def pair(i, carry):
        carry = step(2 * i, svm_a, dvm_a, isem_a, carry)
        carry = step(2 * i + 1, svm_b, dvm_b, isem_b, carry)
        return carry

    sp, nf = lax.fori_loop(0, NBLK // 2, pair, (jnp.int32(0), jnp.int32(0)))

    ti = TRASH + lax.broadcasted_iota(jnp.int32, (16,), 0)
    zv = jnp.zeros((16,), jnp.int32)

    def pbody(j, sp):
        s_st[pl.ds(sp, 16)] = zv
        l_st[pl.ds(sp, 16)] = ti
        return sp + 16

    lax.fori_loop(0, BLK // 16, pbody, sp)
    flush(nf)
    nf = nf + 1
    for j in range(BLK // 16):
        s_st[pl.ds(16 * j, 16)] = zv
        l_st[pl.ds(16 * j, 16)] = ti
    odd = (nf & 1) == 1

    @pl.when(odd)
    def _():
        flush(nf)

    nb_final = nf + jnp.where(odd, 1, 0)
    cvm[...] = jnp.full((16,), 1, jnp.int32) * nb_final
    pltpu.sync_copy(cvm, cnts.at[pl.ds(w * 16, 16)])


_bin_call = pl.kernel(
    _bin_body,
    out_type=[
        jax.ShapeDtypeStruct((NW * ETile,), jnp.int32),
        jax.ShapeDtypeStruct((NW * ETile,), jnp.int32),
        jax.ShapeDtypeStruct((NW * 16,), jnp.int32),
    ],
    mesh=_mesh,
    scratch_types=[
        pltpu.VMEM((KR, 128), jnp.int32),
        pltpu.VMEM((KR, 128), jnp.int32),
        pltpu.VMEM((KR, 128), jnp.int32),
        pltpu.VMEM((KR, 128), jnp.int32),
        pltpu.VMEM((2 * BLK,), jnp.int32),
        pltpu.VMEM((2 * BLK,), jnp.int32),
        pltpu.VMEM((16,), jnp.int32),
        pltpu.SemaphoreType.DMA,
        pltpu.SemaphoreType.DMA,
    ],
    compiler_params=_sc_params,
)


def _agg_body(z, sbin, lbin, cnts, agg_out,
              svm_a, svm_b, lvm_a, lvm_b, rows_a, rows_b, cvm,
              isem_a, isem_b, gsem_a, gsem_b, acc_sh):
    c = lax.axis_index("c")
    s = lax.axis_index("s")
    w = c * NS + s
    rbase = w * BRows

    def zbody(r, _):
        z16 = jnp.zeros((16,), jnp.float32)
        rows_a[r, pl.ds(0, 16)] = z16
        rows_a[r, pl.ds(16, 16)] = z16
        return 0

    lax.fori_loop(0, WCH, zbody, 0)
    for q in range(ROWS_T // WCH):
        pltpu.sync_copy(
            rows_a.at[pl.ds(0, WCH)], acc_sh.at[pl.ds(s * ROWS_T + q * WCH, WCH)]
        )
    plsc.subcore_barrier()

    pltpu.sync_copy(cnts.at[pl.ds(w * 16, 16)], cvm)
    nb = lax.reduce_max(cvm[...], (0,))

    def wait_idx(svm, lvm, isem):
        pltpu.make_async_copy(sbin.at[pl.ds(rbase, KR)], svm, isem).wait()
        pltpu.make_async_copy(lbin.at[pl.ds(rbase, KR)], lvm, isem).wait()

    def fire_gathers(svm, rows, gsem):
        return [
            pltpu.async_copy(z.at[svm.at[k]], rows.at[pl.ds(k * 128, 128)], gsem)
            for k in range(KR)
        ]

    def fire_src(b, svm, isem):
        @pl.when(b < nb)
        def _():
            pltpu.async_copy(sbin.at[pl.ds(rbase + b * KR, KR)], svm, isem)

    def fire_ldst(b, lvm, isem):
        @pl.when(b < nb)
        def _():
            pltpu.async_copy(lbin.at[pl.ds(rbase + b * KR, KR)], lvm, isem)

    def scatter(lvm, rows):
        for k in range(KR):
            pltpu.sync_copy(
                rows.at[pl.ds(k * 128, 128)], acc_sh.at[lvm.at[k]], add=True
            )

    def pair(i, _):
        b0 = 2 * i
        wait_idx(svm_a, lvm_a, isem_a)
        ga = fire_gathers(svm_a, rows_a, gsem_a)
        wait_idx(svm_b, lvm_b, isem_b)
        gb = fire_gathers(svm_b, rows_b, gsem_b)
        for d in ga:
            d.wait()
        fire_src(b0 + 2, svm_a, isem_a)
        scatter(lvm_a, rows_a)
        fire_ldst(b0 + 2, lvm_a, isem_a)
        for d in gb:
            d.wait()
        fire_src(b0 + 3, svm_b, isem_b)
        scatter(lvm_b, rows_b)
        fire_ldst(b0 + 3, lvm_b, isem_b)
        return 0

    pltpu.async_copy(sbin.at[pl.ds(rbase, KR)], svm_a, isem_a)
    pltpu.async_copy(lbin.at[pl.ds(rbase, KR)], lvm_a, isem_a)
    pltpu.async_copy(sbin.at[pl.ds(rbase + KR, KR)], svm_b, isem_b)
    pltpu.async_copy(lbin.at[pl.ds(rbase + KR, KR)], lvm_b, isem_b)
    lax.fori_loop(0, lax.shift_right_logical(nb, 1), pair, 0)

    plsc.subcore_barrier()
    for q in range(ROWS_T // WCH):
        base = s * ROWS_T + q * WCH
        pltpu.sync_copy(acc_sh.at[pl.ds(base, WCH)], rows_a.at[pl.ds(0, WCH)])
        pltpu.sync_copy(
            rows_a.at[pl.ds(0, WCH)], agg_out.at[pl.ds(c * SPADR + base, WCH)]
        )


_agg_call = pl.kernel(
    _agg_body,
    out_type=jax.ShapeDtypeStruct((NC * SPADR, H), jnp.float32),
    mesh=_mesh,
    scratch_types=[
        pltpu.VMEM((KR, 128), jnp.int32),
        pltpu.VMEM((KR, 128), jnp.int32),
        pltpu.VMEM((KR, 128), jnp.int32),
        pltpu.VMEM((KR, 128), jnp.int32),
        pltpu.VMEM((BLK, H), jnp.float32),
        pltpu.VMEM((BLK, H), jnp.float32),
        pltpu.VMEM((16,), jnp.int32),
        pltpu.SemaphoreType.DMA,
        pltpu.SemaphoreType.DMA,
        pltpu.SemaphoreType.DMA,
        pltpu.SemaphoreType.DMA,
        pltpu.VMEM_SHARED((SPADR, H), jnp.float32),
    ],
    compiler_params=_sc_params,
)


_RB = 1000
_GRID = N // _RB


def _dense0_body(deg_ref, x_ref, w_ref, dinv_ref, z_ref):
    dinv = lax.rsqrt(deg_ref[...] + 1.0)
    dinv_ref[...] = dinv
    z_ref[...] = dinv * jnp.dot(
        x_ref[...], w_ref[...], preferred_element_type=jnp.float32
    )


_dense0 = pl.pallas_call(
    _dense0_body,
    grid=(_GRID,),
    in_specs=[
        pl.BlockSpec((_RB, 1), lambda i: (i, 0)),
        pl.BlockSpec((_RB, 2), lambda i: (i, 0)),
        pl.BlockSpec((2, H), lambda i: (0, 0)),
    ],
    out_specs=[
        pl.BlockSpec((_RB, 1), lambda i: (i, 0)),
        pl.BlockSpec((_RB, H), lambda i: (i, 0)),
    ],
    out_shape=[
        jax.ShapeDtypeStruct((N, 1), jnp.float32),
        jax.ShapeDtypeStruct((N, H), jnp.float32),
    ],
)


def _dense_mid_body(agg_ref, z_ref, dinv_ref, b_ref, w_ref, zn_ref):
    dinv = dinv_ref[...]
    h = jnp.maximum(dinv * (agg_ref[...] + z_ref[...]) + b_ref[...], 0.0)
    zn_ref[...] = dinv * jnp.dot(
        h, w_ref[...], preferred_element_type=jnp.float32
    )


_dense_mid = pl.pallas_call(
    _dense_mid_body,
    grid=(_GRID,),
    in_specs=[
        pl.BlockSpec((_RB, H), lambda i: (i, 0)),
        pl.BlockSpec((_RB, H), lambda i: (i, 0)),
        pl.BlockSpec((_RB, 1), lambda i: (i, 0)),
        pl.BlockSpec((1, H), lambda i: (0, 0)),
        pl.BlockSpec((H, H), lambda i: (0, 0)),
    ],
    out_specs=pl.BlockSpec((_RB, H), lambda i: (i, 0)),
    out_shape=jax.ShapeDtypeStruct((N, H), jnp.float32),
)


def _dense_fin_body(agg_ref, z_ref, dinv_ref, b_ref, wl_ref, bl_ref, out_ref):
    dinv = dinv_ref[...]
    h = jnp.maximum(dinv * (agg_ref[...] + z_ref[...]) + b_ref[...], 0.0)
    out_ref[...] = (
        jnp.dot(h, wl_ref[...], preferred_element_type=jnp.float32) + bl_ref[...]
    )


_dense_fin = pl.pallas_call(
    _dense_fin_body,
    grid=(_GRID,),
    in_specs=[
        pl.BlockSpec((_RB, H), lambda i: (i, 0)),
        pl.BlockSpec((_RB, H), lambda i: (i, 0)),
        pl.BlockSpec((_RB, 1), lambda i: (i, 0)),
        pl.BlockSpec((1, H), lambda i: (0, 0)),
        pl.BlockSpec((H, 1), lambda i: (0, 0)),
        pl.BlockSpec((1, 1), lambda i: (0, 0)),
    ],
    out_specs=pl.BlockSpec((_RB, 1), lambda i: (i, 0)),
    out_shape=jax.ShapeDtypeStruct((N, 1), jnp.float32),
)


def _unpad(a):
    return jnp.concatenate([a[:NHALF], a[SPADR : SPADR + NHALF]], axis=0)


@jax.jit
def kernel(x, edge_index, W1, b1, W2, b2, W3, b3, Wl, bl):
    src = edge_index[0]
    dst = edge_index[1]
    pad = EPAD - E
    srcp = jnp.concatenate([src, jnp.zeros((pad,), jnp.int32)]).reshape(ERows, 128)
    dstp = jnp.concatenate([dst, jnp.full((pad,), N, jnp.int32)]).reshape(ERows, 128)

    sbin, lbin, cnts = _bin_call(srcp, dstp)
    sbin = sbin.reshape(NW * BRows, 128)
    lbin = lbin.reshape(NW * BRows, 128)

    degp = _deg_call(dstp)
    deg = _unpad(degp)[:, None]

    dinv, z1 = _dense0(deg, x, W1)
    a1 = _unpad(_agg_call(z1, sbin, lbin, cnts))
    z2 = _dense_mid(a1, z1, dinv, b1[None, :], W2)
    a2 = _unpad(_agg_call(z2, sbin, lbin, cnts))
    z3 = _dense_mid(a2, z2, dinv, b2[None, :], W3)
    a3 = _unpad(_agg_call(z3, sbin, lbin, cnts))
    return _dense_fin(a3, z3, dinv, b3[None, :], Wl, bl[None, :])
